# baseline (speedup 1.0000x reference)
import jax
import jax.numpy as jnp
import numpy as np
from jax.experimental import pallas as pl
from jax.experimental.pallas import tpu as pltpu

_N, _D, _AH, _CH, _A = 10000, 128, 256, 256, 100000
_BA = 4096
_NSTEPS = -(-_A // _BA)


def _np_threefry2x32(k1, k2, x0, x1):
    rot = (np.uint32(13), np.uint32(15), np.uint32(26), np.uint32(6),
           np.uint32(17), np.uint32(29), np.uint32(16), np.uint32(24))

    def rotl(x, d):
        return (x << d) | (x >> np.uint32(32 - int(d)))

    ks = [np.uint32(k1), np.uint32(k2),
          np.uint32(k1) ^ np.uint32(k2) ^ np.uint32(0x1BD11BDA)]
    x0 = x0 + ks[0]
    x1 = x1 + ks[1]
    for i in range(5):
        rots = rot[:4] if i % 2 == 0 else rot[4:]
        for r in rots:
            x0 = x0 + x1
            x1 = rotl(x1, r)
            x1 = x0 ^ x1
        x0 = x0 + ks[(i + 1) % 3]
        x1 = x1 + ks[(i + 2) % 3] + np.uint32(i + 1)
    return x0, x1


def _np_gumbel(seed, n):
    k1, k2 = np.uint32(seed >> 32), np.uint32(seed & 0xFFFFFFFF)
    iota = np.arange(n, dtype=np.uint64)
    c1 = (iota >> np.uint64(32)).astype(np.uint32)
    c2 = (iota & np.uint64(0xFFFFFFFF)).astype(np.uint32)
    b1, b2 = _np_threefry2x32(k1, k2, c1, c2)
    bits = b1 ^ b2
    float_bits = (bits >> np.uint32(9)) | np.uint32(0x3F800000)
    floats = float_bits.view(np.float32) - np.float32(1.0)
    tiny = np.float32(np.finfo(np.float32).tiny)
    u = np.maximum(tiny, floats * (np.float32(1.0) - tiny) + tiny)
    with np.errstate(divide="ignore"):
        return -np.log(-np.log(u))


_G_NODE = _np_gumbel(42, _N).reshape(_N, 1)
_G_XFER = _np_gumbel(43, _A).reshape(1, _A)


_IOTA_N = np.arange(_N, dtype=np.int32).reshape(_N, 1)
_IOTA_BA = np.arange(_BA, dtype=np.int32).reshape(1, _BA)


def _critic_body(ge, g42, iota_n, v1, c1, v2, c2, w1, b1, node_o, xh_o):
    h = jnp.maximum(
        jnp.dot(ge[...], v1[...], preferred_element_type=jnp.float32) + c1[...], 0.0)
    vs = jnp.dot(h, v2[...], preferred_element_type=jnp.float32) + c2[...]
    score_n = vs + g42[...]
    m = jnp.max(score_n)
    node = jnp.min(jnp.where(score_n == m, iota_n[...], _N))
    node_o[0] = node
    x = ge[pl.ds(node, 1), :]
    xh_o[...] = jnp.maximum(
        jnp.dot(x, w1[...], preferred_element_type=jnp.float32) + b1[...], 0.0)


def _actor_body(xh, iota_c, w2, madd, sadd, xfer_o, lp_o, acc_s, idx_s):
    i = pl.program_id(0)

    @pl.when(i == 0)
    def _init():
        acc_s[0] = -jnp.inf
        acc_s[1] = 0.0
        acc_s[2] = -jnp.inf
        acc_s[3] = 0.0
        idx_s[0] = 0

    mm = jnp.dot(xh[...], w2[...], preferred_element_type=jnp.float32)
    masked = mm + madd[...]
    score = mm + sadd[...]

    m_old = acc_s[2]
    m_new = jnp.maximum(m_old, jnp.max(masked))
    acc_s[3] = acc_s[3] * jnp.exp(m_old - m_new) + jnp.sum(jnp.exp(masked - m_new))
    acc_s[2] = m_new

    bm = jnp.max(score)

    @pl.when(bm > acc_s[0])
    def _update_best():
        loc = jnp.min(jnp.where(score == bm, iota_c[...], _BA))
        acc_s[0] = bm
        acc_s[1] = jnp.max(jnp.where(iota_c[...] == loc, masked, -jnp.inf))
        idx_s[0] = i * _BA + loc

    @pl.when(i == _NSTEPS - 1)
    def _finalize():
        xfer_o[0] = idx_s[0]
        lp_o[0] = acc_s[1] - (acc_s[2] + jnp.log(acc_s[3]))


def kernel(graph_embed, mask, W1, b1, W2, b2, V1, c1, V2, c2):
    apad = _BA * _NSTEPS
    pen = b2 + jnp.where(mask, jnp.float32(0), jnp.float32(-1e10))
    madd = jnp.concatenate(
        [pen, jnp.full((apad - _A,), -1e30, jnp.float32)]).reshape(1, apad)
    sadd = jnp.concatenate(
        [pen + jnp.asarray(_G_XFER[0]), jnp.full((apad - _A,), -jnp.inf, jnp.float32)]
    ).reshape(1, apad)

    node, xh = pl.pallas_call(
        _critic_body,
        in_specs=[
            pl.BlockSpec((_N, _D), lambda: (0, 0)),
            pl.BlockSpec((_N, 1), lambda: (0, 0)),
            pl.BlockSpec((_N, 1), lambda: (0, 0)),
            pl.BlockSpec((_D, _CH), lambda: (0, 0)),
            pl.BlockSpec((1, _CH), lambda: (0, 0)),
            pl.BlockSpec((_CH, 1), lambda: (0, 0)),
            pl.BlockSpec((1, 1), lambda: (0, 0)),
            pl.BlockSpec((_D, _AH), lambda: (0, 0)),
            pl.BlockSpec((1, _AH), lambda: (0, 0)),
        ],
        out_specs=[
            pl.BlockSpec(memory_space=pltpu.SMEM),
            pl.BlockSpec((1, _AH), lambda: (0, 0)),
        ],
        out_shape=[
            jax.ShapeDtypeStruct((1,), jnp.int32),
            jax.ShapeDtypeStruct((1, _AH), jnp.float32),
        ],
    )(graph_embed, jnp.asarray(_G_NODE), jnp.asarray(_IOTA_N),
      V1, c1.reshape(1, _CH), V2, c2.reshape(1, 1), W1, b1.reshape(1, _AH))

    xfer, lp = pl.pallas_call(
        _actor_body,
        grid=(_NSTEPS,),
        in_specs=[
            pl.BlockSpec((1, _AH), lambda i: (0, 0)),
            pl.BlockSpec((1, _BA), lambda i: (0, 0)),
            pl.BlockSpec((_AH, _BA), lambda i: (0, i)),
            pl.BlockSpec((1, _BA), lambda i: (0, i)),
            pl.BlockSpec((1, _BA), lambda i: (0, i)),
        ],
        out_specs=[
            pl.BlockSpec(memory_space=pltpu.SMEM),
            pl.BlockSpec(memory_space=pltpu.SMEM),
        ],
        out_shape=[
            jax.ShapeDtypeStruct((1,), jnp.int32),
            jax.ShapeDtypeStruct((1,), jnp.float32),
        ],
        scratch_shapes=[
            pltpu.SMEM((4,), jnp.float32),
            pltpu.SMEM((1,), jnp.int32),
        ],
    )(xh, jnp.asarray(_IOTA_BA), W2, madd, sadd)
    return node.reshape(()), xfer.reshape(()), lp.reshape(())


# speedup vs baseline: 2.5939x; 2.5939x over previous
import jax
import jax.numpy as jnp
import numpy as np
from jax.experimental import pallas as pl
from jax.experimental.pallas import tpu as pltpu

_N, _D, _AH, _CH, _A = 10000, 128, 256, 256, 100000
_BA = 4096
_NSTEPS = -(-_A // _BA)


def _np_threefry2x32(k1, k2, x0, x1):
    rot = (np.uint32(13), np.uint32(15), np.uint32(26), np.uint32(6),
           np.uint32(17), np.uint32(29), np.uint32(16), np.uint32(24))

    def rotl(x, d):
        return (x << d) | (x >> np.uint32(32 - int(d)))

    ks = [np.uint32(k1), np.uint32(k2),
          np.uint32(k1) ^ np.uint32(k2) ^ np.uint32(0x1BD11BDA)]
    x0 = x0 + ks[0]
    x1 = x1 + ks[1]
    for i in range(5):
        rots = rot[:4] if i % 2 == 0 else rot[4:]
        for r in rots:
            x0 = x0 + x1
            x1 = rotl(x1, r)
            x1 = x0 ^ x1
        x0 = x0 + ks[(i + 1) % 3]
        x1 = x1 + ks[(i + 2) % 3] + np.uint32(i + 1)
    return x0, x1


def _np_gumbel(seed, n):
    k1, k2 = np.uint32(seed >> 32), np.uint32(seed & 0xFFFFFFFF)
    iota = np.arange(n, dtype=np.uint64)
    c1 = (iota >> np.uint64(32)).astype(np.uint32)
    c2 = (iota & np.uint64(0xFFFFFFFF)).astype(np.uint32)
    b1, b2 = _np_threefry2x32(k1, k2, c1, c2)
    bits = b1 ^ b2
    float_bits = (bits >> np.uint32(9)) | np.uint32(0x3F800000)
    floats = float_bits.view(np.float32) - np.float32(1.0)
    tiny = np.float32(np.finfo(np.float32).tiny)
    u = np.maximum(tiny, floats * (np.float32(1.0) - tiny) + tiny)
    with np.errstate(divide="ignore"):
        return -np.log(-np.log(u))


_G_NODE = _np_gumbel(42, _N).reshape(_N, 1)
_G_XFER = _np_gumbel(43, _A).reshape(1, _A)


_IOTA_N = np.arange(_N, dtype=np.int32).reshape(_N, 1)
_IOTA_BA = np.arange(_BA, dtype=np.int32).reshape(1, _BA)


def _critic_body(ge, g42, iota_n, v1, c1, v2r, c2, w1, b1, node_o, xh_o):
    h = jnp.maximum(
        jnp.dot(ge[...], v1[...], preferred_element_type=jnp.float32) + c1[...], 0.0)
    vs = jnp.sum(h * v2r[...], axis=1, keepdims=True) + c2[...]
    score_n = vs + g42[...]
    m = jnp.max(score_n)
    node = jnp.min(jnp.where(score_n == m, iota_n[...], _N))
    node_o[0] = node
    x = ge[pl.ds(node, 1), :]
    xh_o[...] = jnp.maximum(
        jnp.dot(x, w1[...], preferred_element_type=jnp.float32) + b1[...], 0.0)


def _actor_body(xh, iota_c, w2t, madd, sadd, xfer_o, lp_o, acc_s, idx_s):
    i = pl.program_id(0)

    @pl.when(i == 0)
    def _init():
        acc_s[0] = -jnp.inf
        acc_s[1] = 0.0
        acc_s[2] = -jnp.inf
        acc_s[3] = 0.0
        idx_s[0] = 0

    mm = jax.lax.dot_general(xh[...], w2t[...],
                             dimension_numbers=(((1,), (1,)), ((), ())),
                             preferred_element_type=jnp.float32)
    masked = mm + madd[...]
    score = mm + sadd[...]

    m_old = acc_s[2]
    m_new = jnp.maximum(m_old, jnp.max(masked))
    acc_s[3] = acc_s[3] * jnp.exp(m_old - m_new) + jnp.sum(jnp.exp(masked - m_new))
    acc_s[2] = m_new

    bm = jnp.max(score)

    @pl.when(bm > acc_s[0])
    def _update_best():
        loc = jnp.min(jnp.where(score == bm, iota_c[...], _BA))
        acc_s[0] = bm
        acc_s[1] = jnp.max(jnp.where(iota_c[...] == loc, masked, -jnp.inf))
        idx_s[0] = i * _BA + loc

    @pl.when(i == _NSTEPS - 1)
    def _finalize():
        xfer_o[0] = idx_s[0]
        lp_o[0] = acc_s[1] - (acc_s[2] + jnp.log(acc_s[3]))


def kernel(graph_embed, mask, W1, b1, W2, b2, V1, c1, V2, c2):
    apad = _BA * _NSTEPS
    pen = b2 + jnp.where(mask, jnp.float32(0), jnp.float32(-1e10))
    madd = jnp.concatenate(
        [pen, jnp.full((apad - _A,), -1e30, jnp.float32)]).reshape(1, apad)
    sadd = jnp.concatenate(
        [pen + jnp.asarray(_G_XFER[0]), jnp.full((apad - _A,), -jnp.inf, jnp.float32)]
    ).reshape(1, apad)

    node, xh = pl.pallas_call(
        _critic_body,
        in_specs=[
            pl.BlockSpec((_N, _D), lambda: (0, 0)),
            pl.BlockSpec((_N, 1), lambda: (0, 0)),
            pl.BlockSpec((_N, 1), lambda: (0, 0)),
            pl.BlockSpec((_D, _CH), lambda: (0, 0)),
            pl.BlockSpec((1, _CH), lambda: (0, 0)),
            pl.BlockSpec((1, _CH), lambda: (0, 0)),
            pl.BlockSpec((1, 1), lambda: (0, 0)),
            pl.BlockSpec((_D, _AH), lambda: (0, 0)),
            pl.BlockSpec((1, _AH), lambda: (0, 0)),
        ],
        out_specs=[
            pl.BlockSpec(memory_space=pltpu.SMEM),
            pl.BlockSpec((1, _AH), lambda: (0, 0)),
        ],
        out_shape=[
            jax.ShapeDtypeStruct((1,), jnp.int32),
            jax.ShapeDtypeStruct((1, _AH), jnp.float32),
        ],
    )(graph_embed, jnp.asarray(_G_NODE), jnp.asarray(_IOTA_N),
      V1, c1.reshape(1, _CH), V2.T, c2.reshape(1, 1), W1, b1.reshape(1, _AH))

    xfer, lp = pl.pallas_call(
        _actor_body,
        grid=(_NSTEPS,),
        in_specs=[
            pl.BlockSpec((1, _AH), lambda i: (0, 0)),
            pl.BlockSpec((1, _BA), lambda i: (0, 0)),
            pl.BlockSpec((_BA, _AH), lambda i: (i, 0)),
            pl.BlockSpec((1, _BA), lambda i: (0, i)),
            pl.BlockSpec((1, _BA), lambda i: (0, i)),
        ],
        out_specs=[
            pl.BlockSpec(memory_space=pltpu.SMEM),
            pl.BlockSpec(memory_space=pltpu.SMEM),
        ],
        out_shape=[
            jax.ShapeDtypeStruct((1,), jnp.int32),
            jax.ShapeDtypeStruct((1,), jnp.float32),
        ],
        scratch_shapes=[
            pltpu.SMEM((4,), jnp.float32),
            pltpu.SMEM((1,), jnp.int32),
        ],
    )(xh, jnp.asarray(_IOTA_BA), W2.T, madd, sadd)
    return node.reshape(()), xfer.reshape(()), lp.reshape(())


# speedup vs baseline: 2.9134x; 1.1232x over previous
import jax
import jax.numpy as jnp
import numpy as np
from jax.experimental import pallas as pl
from jax.experimental.pallas import tpu as pltpu

_N, _D, _AH, _CH, _A = 10000, 128, 256, 256, 100000
_BA = 8192
_NSTEPS = -(-_A // _BA)


def _np_threefry2x32(k1, k2, x0, x1):
    rot = (np.uint32(13), np.uint32(15), np.uint32(26), np.uint32(6),
           np.uint32(17), np.uint32(29), np.uint32(16), np.uint32(24))

    def rotl(x, d):
        return (x << d) | (x >> np.uint32(32 - int(d)))

    ks = [np.uint32(k1), np.uint32(k2),
          np.uint32(k1) ^ np.uint32(k2) ^ np.uint32(0x1BD11BDA)]
    x0 = x0 + ks[0]
    x1 = x1 + ks[1]
    for i in range(5):
        rots = rot[:4] if i % 2 == 0 else rot[4:]
        for r in rots:
            x0 = x0 + x1
            x1 = rotl(x1, r)
            x1 = x0 ^ x1
        x0 = x0 + ks[(i + 1) % 3]
        x1 = x1 + ks[(i + 2) % 3] + np.uint32(i + 1)
    return x0, x1


def _np_gumbel(seed, n):
    k1, k2 = np.uint32(seed >> 32), np.uint32(seed & 0xFFFFFFFF)
    iota = np.arange(n, dtype=np.uint64)
    c1 = (iota >> np.uint64(32)).astype(np.uint32)
    c2 = (iota & np.uint64(0xFFFFFFFF)).astype(np.uint32)
    b1, b2 = _np_threefry2x32(k1, k2, c1, c2)
    bits = b1 ^ b2
    float_bits = (bits >> np.uint32(9)) | np.uint32(0x3F800000)
    floats = float_bits.view(np.float32) - np.float32(1.0)
    tiny = np.float32(np.finfo(np.float32).tiny)
    u = np.maximum(tiny, floats * (np.float32(1.0) - tiny) + tiny)
    with np.errstate(divide="ignore"):
        return -np.log(-np.log(u))


_G_NODE = _np_gumbel(42, _N).reshape(_N, 1)
_G_XFER = _np_gumbel(43, _A).reshape(1, _A)


_IOTA_N = np.arange(_N, dtype=np.int32).reshape(_N, 1)
_IOTA_BA = np.arange(_BA, dtype=np.int32).reshape(1, _BA)


def _critic_body(ge, g42, iota_n, v1, c1, v2r, c2, w1, b1, node_o, xh_o):
    h = jnp.maximum(
        jnp.dot(ge[...], v1[...], preferred_element_type=jnp.float32) + c1[...], 0.0)
    vs = jnp.sum(h * v2r[...], axis=1, keepdims=True) + c2[...]
    score_n = vs + g42[...]
    m = jnp.max(score_n)
    node = jnp.min(jnp.where(score_n == m, iota_n[...], _N))
    node_o[0] = node
    x = ge[pl.ds(node, 1), :]
    xh_o[...] = jnp.maximum(
        jnp.dot(x, w1[...], preferred_element_type=jnp.float32) + b1[...], 0.0)


def _actor_body(xh, iota_c, w2t, madd, sadd, xfer_o, lp_o, acc_s, idx_s):
    i = pl.program_id(0)

    @pl.when(i == 0)
    def _init():
        acc_s[0] = -jnp.inf
        acc_s[1] = 0.0
        acc_s[2] = -jnp.inf
        acc_s[3] = 0.0
        idx_s[0] = 0

    mm = jax.lax.dot_general(xh[...], w2t[...],
                             dimension_numbers=(((1,), (1,)), ((), ())),
                             preferred_element_type=jnp.float32)
    masked = mm + madd[...]
    score = mm + sadd[...]

    m_old = acc_s[2]
    m_new = jnp.maximum(m_old, jnp.max(masked))
    acc_s[3] = acc_s[3] * jnp.exp(m_old - m_new) + jnp.sum(jnp.exp(masked - m_new))
    acc_s[2] = m_new

    bm = jnp.max(score)

    @pl.when(bm > acc_s[0])
    def _update_best():
        loc = jnp.min(jnp.where(score == bm, iota_c[...], _BA))
        acc_s[0] = bm
        acc_s[1] = jnp.max(jnp.where(iota_c[...] == loc, masked, -jnp.inf))
        idx_s[0] = i * _BA + loc

    @pl.when(i == _NSTEPS - 1)
    def _finalize():
        xfer_o[0] = idx_s[0]
        lp_o[0] = acc_s[1] - (acc_s[2] + jnp.log(acc_s[3]))


def kernel(graph_embed, mask, W1, b1, W2, b2, V1, c1, V2, c2):
    apad = _BA * _NSTEPS
    pen = b2 + jnp.where(mask, jnp.float32(0), jnp.float32(-1e10))
    madd = jnp.concatenate(
        [pen, jnp.full((apad - _A,), -1e30, jnp.float32)]).reshape(1, apad)
    sadd = jnp.concatenate(
        [pen + jnp.asarray(_G_XFER[0]), jnp.full((apad - _A,), -jnp.inf, jnp.float32)]
    ).reshape(1, apad)

    node, xh = pl.pallas_call(
        _critic_body,
        in_specs=[
            pl.BlockSpec((_N, _D), lambda: (0, 0)),
            pl.BlockSpec((_N, 1), lambda: (0, 0)),
            pl.BlockSpec((_N, 1), lambda: (0, 0)),
            pl.BlockSpec((_D, _CH), lambda: (0, 0)),
            pl.BlockSpec((1, _CH), lambda: (0, 0)),
            pl.BlockSpec((1, _CH), lambda: (0, 0)),
            pl.BlockSpec((1, 1), lambda: (0, 0)),
            pl.BlockSpec((_D, _AH), lambda: (0, 0)),
            pl.BlockSpec((1, _AH), lambda: (0, 0)),
        ],
        out_specs=[
            pl.BlockSpec(memory_space=pltpu.SMEM),
            pl.BlockSpec((1, _AH), lambda: (0, 0)),
        ],
        out_shape=[
            jax.ShapeDtypeStruct((1,), jnp.int32),
            jax.ShapeDtypeStruct((1, _AH), jnp.float32),
        ],
    )(graph_embed, jnp.asarray(_G_NODE), jnp.asarray(_IOTA_N),
      V1, c1.reshape(1, _CH), V2.T, c2.reshape(1, 1), W1, b1.reshape(1, _AH))

    xfer, lp = pl.pallas_call(
        _actor_body,
        grid=(_NSTEPS,),
        in_specs=[
            pl.BlockSpec((1, _AH), lambda i: (0, 0)),
            pl.BlockSpec((1, _BA), lambda i: (0, 0)),
            pl.BlockSpec((_BA, _AH), lambda i: (i, 0)),
            pl.BlockSpec((1, _BA), lambda i: (0, i)),
            pl.BlockSpec((1, _BA), lambda i: (0, i)),
        ],
        out_specs=[
            pl.BlockSpec(memory_space=pltpu.SMEM),
            pl.BlockSpec(memory_space=pltpu.SMEM),
        ],
        out_shape=[
            jax.ShapeDtypeStruct((1,), jnp.int32),
            jax.ShapeDtypeStruct((1,), jnp.float32),
        ],
        scratch_shapes=[
            pltpu.SMEM((4,), jnp.float32),
            pltpu.SMEM((1,), jnp.int32),
        ],
    )(xh, jnp.asarray(_IOTA_BA), W2.T, madd, sadd)
    return node.reshape(()), xfer.reshape(()), lp.reshape(())


# speedup vs baseline: 2.9356x; 1.0076x over previous
import jax
import jax.numpy as jnp
import numpy as np
from jax.experimental import pallas as pl
from jax.experimental.pallas import tpu as pltpu

_N, _D, _AH, _CH, _A = 10000, 128, 256, 256, 100000
_BA = 12288
_NSTEPS = -(-_A // _BA)


def _np_threefry2x32(k1, k2, x0, x1):
    rot = (np.uint32(13), np.uint32(15), np.uint32(26), np.uint32(6),
           np.uint32(17), np.uint32(29), np.uint32(16), np.uint32(24))

    def rotl(x, d):
        return (x << d) | (x >> np.uint32(32 - int(d)))

    ks = [np.uint32(k1), np.uint32(k2),
          np.uint32(k1) ^ np.uint32(k2) ^ np.uint32(0x1BD11BDA)]
    x0 = x0 + ks[0]
    x1 = x1 + ks[1]
    for i in range(5):
        rots = rot[:4] if i % 2 == 0 else rot[4:]
        for r in rots:
            x0 = x0 + x1
            x1 = rotl(x1, r)
            x1 = x0 ^ x1
        x0 = x0 + ks[(i + 1) % 3]
        x1 = x1 + ks[(i + 2) % 3] + np.uint32(i + 1)
    return x0, x1


def _np_gumbel(seed, n):
    k1, k2 = np.uint32(seed >> 32), np.uint32(seed & 0xFFFFFFFF)
    iota = np.arange(n, dtype=np.uint64)
    c1 = (iota >> np.uint64(32)).astype(np.uint32)
    c2 = (iota & np.uint64(0xFFFFFFFF)).astype(np.uint32)
    b1, b2 = _np_threefry2x32(k1, k2, c1, c2)
    bits = b1 ^ b2
    float_bits = (bits >> np.uint32(9)) | np.uint32(0x3F800000)
    floats = float_bits.view(np.float32) - np.float32(1.0)
    tiny = np.float32(np.finfo(np.float32).tiny)
    u = np.maximum(tiny, floats * (np.float32(1.0) - tiny) + tiny)
    with np.errstate(divide="ignore"):
        return -np.log(-np.log(u))


_G_NODE = _np_gumbel(42, _N).reshape(_N, 1)
_G_XFER = _np_gumbel(43, _A).reshape(1, _A)


_IOTA_N = np.arange(_N, dtype=np.int32).reshape(_N, 1)
_IOTA_BA = np.arange(_BA, dtype=np.int32).reshape(1, _BA)


def _critic_body(ge, g42, iota_n, v1, c1, v2r, c2, w1, b1, node_o, xh_o):
    h = jnp.maximum(
        jnp.dot(ge[...], v1[...], preferred_element_type=jnp.float32) + c1[...], 0.0)
    vs = jnp.sum(h * v2r[...], axis=1, keepdims=True) + c2[...]
    score_n = vs + g42[...]
    m = jnp.max(score_n)
    node = jnp.min(jnp.where(score_n == m, iota_n[...], _N))
    node_o[0] = node
    x = ge[pl.ds(node, 1), :]
    xh_o[...] = jnp.maximum(
        jnp.dot(x, w1[...], preferred_element_type=jnp.float32) + b1[...], 0.0)


def _actor_body(xh, iota_c, w2t, madd, sadd, xfer_o, lp_o, acc_s, idx_s):
    i = pl.program_id(0)

    @pl.when(i == 0)
    def _init():
        acc_s[0] = -jnp.inf
        acc_s[1] = 0.0
        acc_s[2] = -jnp.inf
        acc_s[3] = 0.0
        idx_s[0] = 0

    mm = jax.lax.dot_general(xh[...], w2t[...],
                             dimension_numbers=(((1,), (1,)), ((), ())),
                             preferred_element_type=jnp.float32)
    masked = mm + madd[...]
    score = mm + sadd[...]

    m_old = acc_s[2]
    m_new = jnp.maximum(m_old, jnp.max(masked))
    acc_s[3] = acc_s[3] * jnp.exp(m_old - m_new) + jnp.sum(jnp.exp(masked - m_new))
    acc_s[2] = m_new

    bm = jnp.max(score)

    @pl.when(bm > acc_s[0])
    def _update_best():
        loc = jnp.min(jnp.where(score == bm, iota_c[...], _BA))
        acc_s[0] = bm
        acc_s[1] = jnp.max(jnp.where(iota_c[...] == loc, masked, -jnp.inf))
        idx_s[0] = i * _BA + loc

    @pl.when(i == _NSTEPS - 1)
    def _finalize():
        xfer_o[0] = idx_s[0]
        lp_o[0] = acc_s[1] - (acc_s[2] + jnp.log(acc_s[3]))


def kernel(graph_embed, mask, W1, b1, W2, b2, V1, c1, V2, c2):
    apad = _BA * _NSTEPS
    pen = b2 + jnp.where(mask, jnp.float32(0), jnp.float32(-1e10))
    madd = jnp.concatenate(
        [pen, jnp.full((apad - _A,), -1e30, jnp.float32)]).reshape(1, apad)
    sadd = jnp.concatenate(
        [pen + jnp.asarray(_G_XFER[0]), jnp.full((apad - _A,), -jnp.inf, jnp.float32)]
    ).reshape(1, apad)

    node, xh = pl.pallas_call(
        _critic_body,
        in_specs=[
            pl.BlockSpec((_N, _D), lambda: (0, 0)),
            pl.BlockSpec((_N, 1), lambda: (0, 0)),
            pl.BlockSpec((_N, 1), lambda: (0, 0)),
            pl.BlockSpec((_D, _CH), lambda: (0, 0)),
            pl.BlockSpec((1, _CH), lambda: (0, 0)),
            pl.BlockSpec((1, _CH), lambda: (0, 0)),
            pl.BlockSpec((1, 1), lambda: (0, 0)),
            pl.BlockSpec((_D, _AH), lambda: (0, 0)),
            pl.BlockSpec((1, _AH), lambda: (0, 0)),
        ],
        out_specs=[
            pl.BlockSpec(memory_space=pltpu.SMEM),
            pl.BlockSpec((1, _AH), lambda: (0, 0)),
        ],
        out_shape=[
            jax.ShapeDtypeStruct((1,), jnp.int32),
            jax.ShapeDtypeStruct((1, _AH), jnp.float32),
        ],
    )(graph_embed, jnp.asarray(_G_NODE), jnp.asarray(_IOTA_N),
      V1, c1.reshape(1, _CH), V2.T, c2.reshape(1, 1), W1, b1.reshape(1, _AH))

    xfer, lp = pl.pallas_call(
        _actor_body,
        grid=(_NSTEPS,),
        in_specs=[
            pl.BlockSpec((1, _AH), lambda i: (0, 0)),
            pl.BlockSpec((1, _BA), lambda i: (0, 0)),
            pl.BlockSpec((_BA, _AH), lambda i: (i, 0)),
            pl.BlockSpec((1, _BA), lambda i: (0, i)),
            pl.BlockSpec((1, _BA), lambda i: (0, i)),
        ],
        out_specs=[
            pl.BlockSpec(memory_space=pltpu.SMEM),
            pl.BlockSpec(memory_space=pltpu.SMEM),
        ],
        out_shape=[
            jax.ShapeDtypeStruct((1,), jnp.int32),
            jax.ShapeDtypeStruct((1,), jnp.float32),
        ],
        scratch_shapes=[
            pltpu.SMEM((4,), jnp.float32),
            pltpu.SMEM((1,), jnp.int32),
        ],
    )(xh, jnp.asarray(_IOTA_BA), W2.T, madd, sadd)
    return node.reshape(()), xfer.reshape(()), lp.reshape(())


# speedup vs baseline: 3.0135x; 1.0266x over previous
import jax
import jax.numpy as jnp
import numpy as np
from jax.experimental import pallas as pl
from jax.experimental.pallas import tpu as pltpu

_N, _D, _AH, _CH, _A = 10000, 128, 256, 256, 100000
_BA = 6272
_NBLK = -(-_A // _BA)
_NSTEPS = _NBLK // 2
_BA2 = 2 * _BA


def _np_threefry2x32(k1, k2, x0, x1):
    rot = (np.uint32(13), np.uint32(15), np.uint32(26), np.uint32(6),
           np.uint32(17), np.uint32(29), np.uint32(16), np.uint32(24))

    def rotl(x, d):
        return (x << d) | (x >> np.uint32(32 - int(d)))

    ks = [np.uint32(k1), np.uint32(k2),
          np.uint32(k1) ^ np.uint32(k2) ^ np.uint32(0x1BD11BDA)]
    x0 = x0 + ks[0]
    x1 = x1 + ks[1]
    for i in range(5):
        rots = rot[:4] if i % 2 == 0 else rot[4:]
        for r in rots:
            x0 = x0 + x1
            x1 = rotl(x1, r)
            x1 = x0 ^ x1
        x0 = x0 + ks[(i + 1) % 3]
        x1 = x1 + ks[(i + 2) % 3] + np.uint32(i + 1)
    return x0, x1


def _np_gumbel(seed, n):
    k1, k2 = np.uint32(seed >> 32), np.uint32(seed & 0xFFFFFFFF)
    iota = np.arange(n, dtype=np.uint64)
    c1 = (iota >> np.uint64(32)).astype(np.uint32)
    c2 = (iota & np.uint64(0xFFFFFFFF)).astype(np.uint32)
    b1, b2 = _np_threefry2x32(k1, k2, c1, c2)
    bits = b1 ^ b2
    float_bits = (bits >> np.uint32(9)) | np.uint32(0x3F800000)
    floats = float_bits.view(np.float32) - np.float32(1.0)
    tiny = np.float32(np.finfo(np.float32).tiny)
    u = np.maximum(tiny, floats * (np.float32(1.0) - tiny) + tiny)
    with np.errstate(divide="ignore"):
        return -np.log(-np.log(u))


_G_NODE = _np_gumbel(42, _N).reshape(_N, 1)
_G_XFER = _np_gumbel(43, _A).reshape(1, _A)


_IOTA_N = np.arange(_N, dtype=np.int32).reshape(_N, 1)
_IOTA_BA = np.arange(_BA2, dtype=np.int32).reshape(1, _BA2)


def _critic_body(ge, g42, iota_n, v1, c1, v2r, c2, w1, b1, node_o, xh_o):
    h = jnp.maximum(
        jnp.dot(ge[...], v1[...], preferred_element_type=jnp.float32) + c1[...], 0.0)
    vs = jnp.sum(h * v2r[...], axis=1, keepdims=True) + c2[...]
    score_n = vs + g42[...]
    m = jnp.max(score_n)
    node = jnp.min(jnp.where(score_n == m, iota_n[...], _N))
    node_o[0] = node
    x = ge[pl.ds(node, 1), :]
    xh_o[...] = jnp.maximum(
        jnp.dot(x, w1[...], preferred_element_type=jnp.float32) + b1[...], 0.0)


def _actor_body(xh, iota_c, w2a, w2b, madd, sadd, xfer_o, lp_o, acc_s, idx_s):
    i = pl.program_id(0)

    @pl.when(i == 0)
    def _init():
        acc_s[0] = -jnp.inf
        acc_s[1] = 0.0
        acc_s[2] = -jnp.inf
        acc_s[3] = 0.0
        idx_s[0] = 0

    dn = (((1,), (1,)), ((), ()))
    mm = jnp.concatenate(
        [jax.lax.dot_general(xh[...], w2a[...], dimension_numbers=dn,
                             preferred_element_type=jnp.float32),
         jax.lax.dot_general(xh[...], w2b[...], dimension_numbers=dn,
                             preferred_element_type=jnp.float32)], axis=1)
    masked = mm + madd[...]
    score = mm + sadd[...]

    m_old = acc_s[2]
    m_new = jnp.maximum(m_old, jnp.max(masked))
    acc_s[3] = acc_s[3] * jnp.exp(m_old - m_new) + jnp.sum(jnp.exp(masked - m_new))
    acc_s[2] = m_new

    bm = jnp.max(score)

    @pl.when(bm > acc_s[0])
    def _update_best():
        loc = jnp.min(jnp.where(score == bm, iota_c[...], _BA2))
        acc_s[0] = bm
        acc_s[1] = jnp.max(jnp.where(iota_c[...] == loc, masked, -jnp.inf))
        idx_s[0] = i * _BA2 + loc

    @pl.when(i == _NSTEPS - 1)
    def _finalize():
        xfer_o[0] = idx_s[0]
        lp_o[0] = acc_s[1] - (acc_s[2] + jnp.log(acc_s[3]))


def kernel(graph_embed, mask, W1, b1, W2, b2, V1, c1, V2, c2):
    apad = _BA * _NBLK
    pen = b2 + jnp.where(mask, jnp.float32(0), jnp.float32(-1e10))
    madd = jnp.concatenate(
        [pen, jnp.full((apad - _A,), -1e30, jnp.float32)]).reshape(1, apad)
    sadd = jnp.concatenate(
        [pen + jnp.asarray(_G_XFER[0]), jnp.full((apad - _A,), -jnp.inf, jnp.float32)]
    ).reshape(1, apad)

    node, xh = pl.pallas_call(
        _critic_body,
        in_specs=[
            pl.BlockSpec((_N, _D), lambda: (0, 0)),
            pl.BlockSpec((_N, 1), lambda: (0, 0)),
            pl.BlockSpec((_N, 1), lambda: (0, 0)),
            pl.BlockSpec((_D, _CH), lambda: (0, 0)),
            pl.BlockSpec((1, _CH), lambda: (0, 0)),
            pl.BlockSpec((1, _CH), lambda: (0, 0)),
            pl.BlockSpec((1, 1), lambda: (0, 0)),
            pl.BlockSpec((_D, _AH), lambda: (0, 0)),
            pl.BlockSpec((1, _AH), lambda: (0, 0)),
        ],
        out_specs=[
            pl.BlockSpec(memory_space=pltpu.SMEM),
            pl.BlockSpec((1, _AH), lambda: (0, 0)),
        ],
        out_shape=[
            jax.ShapeDtypeStruct((1,), jnp.int32),
            jax.ShapeDtypeStruct((1, _AH), jnp.float32),
        ],
    )(graph_embed, jnp.asarray(_G_NODE), jnp.asarray(_IOTA_N),
      V1, c1.reshape(1, _CH), V2.T, c2.reshape(1, 1), W1, b1.reshape(1, _AH))

    xfer, lp = pl.pallas_call(
        _actor_body,
        grid=(_NSTEPS,),
        in_specs=[
            pl.BlockSpec((1, _AH), lambda i: (0, 0)),
            pl.BlockSpec((1, _BA2), lambda i: (0, 0)),
            pl.BlockSpec((_BA, _AH), lambda i: (2 * i, 0)),
            pl.BlockSpec((_BA, _AH), lambda i: (2 * i + 1, 0)),
            pl.BlockSpec((1, _BA2), lambda i: (0, i)),
            pl.BlockSpec((1, _BA2), lambda i: (0, i)),
        ],
        out_specs=[
            pl.BlockSpec(memory_space=pltpu.SMEM),
            pl.BlockSpec(memory_space=pltpu.SMEM),
        ],
        out_shape=[
            jax.ShapeDtypeStruct((1,), jnp.int32),
            jax.ShapeDtypeStruct((1,), jnp.float32),
        ],
        scratch_shapes=[
            pltpu.SMEM((4,), jnp.float32),
            pltpu.SMEM((1,), jnp.int32),
        ],
    )(xh, jnp.asarray(_IOTA_BA), W2.T, W2.T, madd, sadd)
    return node.reshape(()), xfer.reshape(()), lp.reshape(())


# speedup vs baseline: 3.0231x; 1.0032x over previous
import jax
import jax.numpy as jnp
import numpy as np
from jax.experimental import pallas as pl
from jax.experimental.pallas import tpu as pltpu

_N, _D, _AH, _CH, _A = 10000, 128, 256, 256, 100000
_BA = 3136
_NBLK = -(-_A // _BA)
_NS = 4
_NSTEPS = _NBLK // _NS
_BA2 = _NS * _BA


def _np_threefry2x32(k1, k2, x0, x1):
    rot = (np.uint32(13), np.uint32(15), np.uint32(26), np.uint32(6),
           np.uint32(17), np.uint32(29), np.uint32(16), np.uint32(24))

    def rotl(x, d):
        return (x << d) | (x >> np.uint32(32 - int(d)))

    ks = [np.uint32(k1), np.uint32(k2),
          np.uint32(k1) ^ np.uint32(k2) ^ np.uint32(0x1BD11BDA)]
    x0 = x0 + ks[0]
    x1 = x1 + ks[1]
    for i in range(5):
        rots = rot[:4] if i % 2 == 0 else rot[4:]
        for r in rots:
            x0 = x0 + x1
            x1 = rotl(x1, r)
            x1 = x0 ^ x1
        x0 = x0 + ks[(i + 1) % 3]
        x1 = x1 + ks[(i + 2) % 3] + np.uint32(i + 1)
    return x0, x1


def _np_gumbel(seed, n):
    k1, k2 = np.uint32(seed >> 32), np.uint32(seed & 0xFFFFFFFF)
    iota = np.arange(n, dtype=np.uint64)
    c1 = (iota >> np.uint64(32)).astype(np.uint32)
    c2 = (iota & np.uint64(0xFFFFFFFF)).astype(np.uint32)
    b1, b2 = _np_threefry2x32(k1, k2, c1, c2)
    bits = b1 ^ b2
    float_bits = (bits >> np.uint32(9)) | np.uint32(0x3F800000)
    floats = float_bits.view(np.float32) - np.float32(1.0)
    tiny = np.float32(np.finfo(np.float32).tiny)
    u = np.maximum(tiny, floats * (np.float32(1.0) - tiny) + tiny)
    with np.errstate(divide="ignore"):
        return -np.log(-np.log(u))


_G_NODE = _np_gumbel(42, _N).reshape(_N, 1)
_G_XFER = _np_gumbel(43, _A).reshape(1, _A)


_IOTA_N = np.arange(_N, dtype=np.int32).reshape(_N, 1)
_IOTA_BA = np.arange(_BA2, dtype=np.int32).reshape(1, _BA2)


def _critic_body(ge, g42, iota_n, v1, c1, v2r, c2, w1, b1, node_o, xh_o):
    h = jnp.maximum(
        jnp.dot(ge[...], v1[...], preferred_element_type=jnp.float32) + c1[...], 0.0)
    vs = jnp.sum(h * v2r[...], axis=1, keepdims=True) + c2[...]
    score_n = vs + g42[...]
    m = jnp.max(score_n)
    node = jnp.min(jnp.where(score_n == m, iota_n[...], _N))
    node_o[0] = node
    x = ge[pl.ds(node, 1), :]
    xh_o[...] = jnp.maximum(
        jnp.dot(x, w1[...], preferred_element_type=jnp.float32) + b1[...], 0.0)


def _actor_body(xh, iota_c, w2a, w2b, w2c, w2d, madd, sadd, xfer_o, lp_o, acc_s, idx_s):
    i = pl.program_id(0)

    @pl.when(i == 0)
    def _init():
        acc_s[0] = -jnp.inf
        acc_s[1] = 0.0
        acc_s[2] = -jnp.inf
        acc_s[3] = 0.0
        idx_s[0] = 0

    dn = (((1,), (1,)), ((), ()))
    mm = jnp.concatenate(
        [jax.lax.dot_general(xh[...], w[...], dimension_numbers=dn,
                             preferred_element_type=jnp.float32)
         for w in (w2a, w2b, w2c, w2d)], axis=1)
    masked = mm + madd[...]
    score = mm + sadd[...]

    m_old = acc_s[2]
    m_new = jnp.maximum(m_old, jnp.max(masked))
    acc_s[3] = acc_s[3] * jnp.exp(m_old - m_new) + jnp.sum(jnp.exp(masked - m_new))
    acc_s[2] = m_new

    bm = jnp.max(score)

    @pl.when(bm > acc_s[0])
    def _update_best():
        loc = jnp.min(jnp.where(score == bm, iota_c[...], _BA2))
        acc_s[0] = bm
        acc_s[1] = jnp.max(jnp.where(iota_c[...] == loc, masked, -jnp.inf))
        idx_s[0] = i * _BA2 + loc

    @pl.when(i == _NSTEPS - 1)
    def _finalize():
        xfer_o[0] = idx_s[0]
        lp_o[0] = acc_s[1] - (acc_s[2] + jnp.log(acc_s[3]))


def kernel(graph_embed, mask, W1, b1, W2, b2, V1, c1, V2, c2):
    apad = _BA * _NBLK
    pen = b2 + jnp.where(mask, jnp.float32(0), jnp.float32(-1e10))
    madd = jnp.concatenate(
        [pen, jnp.full((apad - _A,), -1e30, jnp.float32)]).reshape(1, apad)
    sadd = jnp.concatenate(
        [pen + jnp.asarray(_G_XFER[0]), jnp.full((apad - _A,), -jnp.inf, jnp.float32)]
    ).reshape(1, apad)

    node, xh = pl.pallas_call(
        _critic_body,
        in_specs=[
            pl.BlockSpec((_N, _D), lambda: (0, 0)),
            pl.BlockSpec((_N, 1), lambda: (0, 0)),
            pl.BlockSpec((_N, 1), lambda: (0, 0)),
            pl.BlockSpec((_D, _CH), lambda: (0, 0)),
            pl.BlockSpec((1, _CH), lambda: (0, 0)),
            pl.BlockSpec((1, _CH), lambda: (0, 0)),
            pl.BlockSpec((1, 1), lambda: (0, 0)),
            pl.BlockSpec((_D, _AH), lambda: (0, 0)),
            pl.BlockSpec((1, _AH), lambda: (0, 0)),
        ],
        out_specs=[
            pl.BlockSpec(memory_space=pltpu.SMEM),
            pl.BlockSpec((1, _AH), lambda: (0, 0)),
        ],
        out_shape=[
            jax.ShapeDtypeStruct((1,), jnp.int32),
            jax.ShapeDtypeStruct((1, _AH), jnp.float32),
        ],
    )(graph_embed, jnp.asarray(_G_NODE), jnp.asarray(_IOTA_N),
      V1, c1.reshape(1, _CH), V2.T, c2.reshape(1, 1), W1, b1.reshape(1, _AH))

    xfer, lp = pl.pallas_call(
        _actor_body,
        grid=(_NSTEPS,),
        in_specs=[
            pl.BlockSpec((1, _AH), lambda i: (0, 0)),
            pl.BlockSpec((1, _BA2), lambda i: (0, 0)),
            pl.BlockSpec((_BA, _AH), lambda i: (4 * i, 0)),
            pl.BlockSpec((_BA, _AH), lambda i: (4 * i + 1, 0)),
            pl.BlockSpec((_BA, _AH), lambda i: (4 * i + 2, 0)),
            pl.BlockSpec((_BA, _AH), lambda i: (4 * i + 3, 0)),
            pl.BlockSpec((1, _BA2), lambda i: (0, i)),
            pl.BlockSpec((1, _BA2), lambda i: (0, i)),
        ],
        out_specs=[
            pl.BlockSpec(memory_space=pltpu.SMEM),
            pl.BlockSpec(memory_space=pltpu.SMEM),
        ],
        out_shape=[
            jax.ShapeDtypeStruct((1,), jnp.int32),
            jax.ShapeDtypeStruct((1,), jnp.float32),
        ],
        scratch_shapes=[
            pltpu.SMEM((4,), jnp.float32),
            pltpu.SMEM((1,), jnp.int32),
        ],
    )(xh, jnp.asarray(_IOTA_BA), W2.T, W2.T, W2.T, W2.T, madd, sadd)
    return node.reshape(()), xfer.reshape(()), lp.reshape(())
